# Initial kernel scaffold; baseline (speedup 1.0000x reference)
import jax
import jax.numpy as jnp
from jax.experimental import pallas as pl


def kernel(X, W, b, edge_index, drop_rate):
    raise NotImplementedError("write your pallas kernel here")


# speedup vs baseline: 4.8944x; 4.8944x over previous
import functools

import jax
import jax.numpy as jnp
from jax import lax
from jax.experimental import pallas as pl
from jax.experimental.pallas import tpu as pltpu
from jax.experimental.pallas import tpu_sc as plsc

N_V = 10000
N_E = 10000
NNZ = 320000
D = 128
DH = D // 2

NC = 2
NS = 16

EPT = NNZ // NS
CH = 80
NCHUNK = EPT // CH

RP = 10240
RPT = RP // NS


def _sc_scatter_pass(table2, src_idx, dst_idx):
  mesh = plsc.VectorSubcoreMesh(
      core_axis_name="c", subcore_axis_name="s", num_cores=NC,
      num_subcores=NS)

  @functools.partial(
      pl.kernel,
      out_type=(
          jax.ShapeDtypeStruct((NC, RP, DH), jnp.float32),
          jax.ShapeDtypeStruct((RP,), jnp.float32),
      ),
      mesh=mesh,
      scratch_types=[
          pltpu.VMEM((NCHUNK, CH), jnp.int32),
          pltpu.VMEM((NCHUNK, CH), jnp.int32),
          pltpu.VMEM((CH, DH), jnp.float32),
          pltpu.VMEM((128, DH), jnp.float32),
          pltpu.VMEM((RPT,), jnp.float32),
          pltpu.VMEM((CH,), jnp.float32),
          pltpu.VMEM_SHARED((RP, DH), jnp.float32),
          pltpu.VMEM_SHARED((RP,), jnp.float32),
          pltpu.SemaphoreType.DMA,
      ],
      compiler_params=pltpu.CompilerParams(use_tc_tiling_on_sc=False),
  )
  def scatter_kernel(table_hbm, src_hbm, dst_hbm, p_out, c_out,
                     sbuf, dbuf, rows, zrow, zcnt, ones,
                     acc_sh, cnt_sh, sem):
    c = lax.axis_index("c")
    s = lax.axis_index("s")
    row0 = s * RPT

    zeros16 = jnp.zeros((16,), jnp.float32)
    ones16 = jnp.full((16,), 1.0, jnp.float32)

    def fill_zrow(i, carry):
      for j in range(DH // 16):
        zrow[i, pl.ds(j * 16, 16)] = zeros16
      return carry
    lax.fori_loop(0, 128, fill_zrow, 0)

    def fill_zcnt(i, carry):
      zcnt[pl.ds(i * 16, 16)] = zeros16
      return carry
    lax.fori_loop(0, RPT // 16, fill_zcnt, 0)

    for j in range(CH // 16):
      ones[pl.ds(j * 16, 16)] = ones16

    for k in range(RPT // 128):
      pltpu.sync_copy(zrow, acc_sh.at[pl.ds(row0 + k * 128, 128)])
    pltpu.sync_copy(zcnt, cnt_sh.at[pl.ds(row0, RPT)])

    plsc.subcore_barrier()

    pltpu.sync_copy(src_hbm.at[s], sbuf)
    pltpu.sync_copy(dst_hbm.at[s], dbuf)

    def body(j, carry):
      pltpu.async_copy(table_hbm.at[c].at[sbuf.at[j]], rows, sem).wait()
      pltpu.sync_copy(rows, acc_sh.at[dbuf.at[j]], add=True)
      return carry
    lax.fori_loop(0, NCHUNK, body, 0)

    @pl.when(c == 0)
    def _():
      def cbody(j, carry):
        pltpu.sync_copy(ones, cnt_sh.at[dbuf.at[j]], add=True)
        return carry
      lax.fori_loop(0, NCHUNK, cbody, 0)

    plsc.subcore_barrier()

    pltpu.sync_copy(acc_sh.at[pl.ds(row0, RPT)],
                    p_out.at[c, pl.ds(row0, RPT)])
    @pl.when(c == 0)
    def _():
      pltpu.sync_copy(cnt_sh.at[pl.ds(row0, RPT)],
                      c_out.at[pl.ds(row0, RPT)])

  return scatter_kernel(table2, src_idx, dst_idx)


def _matmul_relu(X, WT, b2):
  blk = 1000

  def mm_body(x_ref, w_ref, b_ref, o_ref):
    h = jnp.dot(x_ref[...], w_ref[...], preferred_element_type=jnp.float32)
    h = jnp.maximum(h + b_ref[...], 0.0)
    o_ref[0] = h[:, :DH]
    o_ref[1] = h[:, DH:]

  return pl.pallas_call(
      mm_body,
      grid=(N_V // blk,),
      in_specs=[
          pl.BlockSpec((blk, D), lambda i: (i, 0)),
          pl.BlockSpec((D, D), lambda i: (0, 0)),
          pl.BlockSpec((1, D), lambda i: (0, 0)),
      ],
      out_specs=pl.BlockSpec((NC, blk, DH), lambda i: (0, i, 0)),
      out_shape=jax.ShapeDtypeStruct((NC, N_V, DH), jnp.float32),
  )(X, WT, b2)


def _combine_mean(P, C, relu, split_out):
  blk = 1024

  def comb_body(p_ref, c_ref, o_ref):
    d = jnp.maximum(c_ref[...], 1.0)
    r0 = p_ref[0] / d[:, None]
    r1 = p_ref[1] / d[:, None]
    if relu:
      r0 = jnp.maximum(r0, 0.0)
      r1 = jnp.maximum(r1, 0.0)
    if split_out:
      o_ref[0] = r0
      o_ref[1] = r1
    else:
      o_ref[:, :DH] = r0
      o_ref[:, DH:] = r1

  if split_out:
    out_spec = pl.BlockSpec((NC, blk, DH), lambda i: (0, i, 0))
    out_shape = jax.ShapeDtypeStruct((NC, RP, DH), jnp.float32)
  else:
    out_spec = pl.BlockSpec((blk, D), lambda i: (i, 0))
    out_shape = jax.ShapeDtypeStruct((RP, D), jnp.float32)

  return pl.pallas_call(
      comb_body,
      grid=(RP // blk,),
      in_specs=[
          pl.BlockSpec((NC, blk, DH), lambda i: (0, i, 0)),
          pl.BlockSpec((blk,), lambda i: (i,)),
      ],
      out_specs=out_spec,
      out_shape=out_shape,
  )(P, C)


def kernel(X, W, b, edge_index, drop_rate):
  v = edge_index[0].astype(jnp.int32).reshape(NS, NCHUNK, CH)
  e = edge_index[1].astype(jnp.int32).reshape(NS, NCHUNK, CH)

  H2 = _matmul_relu(X, W.T, b.reshape(1, D))

  P1, C1 = _sc_scatter_pass(H2, v, e)
  Xe2 = _combine_mean(P1, C1, relu=False, split_out=True)

  P2, C2 = _sc_scatter_pass(Xe2, e, v)
  out = _combine_mean(P2, C2, relu=True, split_out=False)
  return out[:N_V]


# speedup vs baseline: 8.8319x; 1.8045x over previous
import functools

import jax
import jax.numpy as jnp
from jax import lax
from jax.experimental import pallas as pl
from jax.experimental.pallas import tpu as pltpu
from jax.experimental.pallas import tpu_sc as plsc

N_V = 10000
N_E = 10000
NNZ = 320000
D = 128
DH = D // 2

NC = 2
NS = 16

EPT = NNZ // NS
CH = 80
NCHUNK = EPT // CH
NBUF = 5

RP = 10240
RPT = RP // NS


def _sc_scatter_pass(table2, src_idx, dst_idx):
  mesh = plsc.VectorSubcoreMesh(
      core_axis_name="c", subcore_axis_name="s", num_cores=NC,
      num_subcores=NS)

  @functools.partial(
      pl.kernel,
      out_type=(
          jax.ShapeDtypeStruct((NC, RP, DH), jnp.float32),
          jax.ShapeDtypeStruct((NC, RP), jnp.float32),
      ),
      mesh=mesh,
      scratch_types=[
          pltpu.VMEM((NCHUNK, CH), jnp.int32),
          pltpu.VMEM((NCHUNK, CH), jnp.int32),
          [pltpu.VMEM((CH, DH), jnp.float32) for _ in range(NBUF)],
          pltpu.VMEM((128, DH), jnp.float32),
          pltpu.VMEM((RPT,), jnp.float32),
          pltpu.VMEM((CH,), jnp.float32),
          pltpu.VMEM_SHARED((RP, DH), jnp.float32),
          pltpu.VMEM_SHARED((RP,), jnp.float32),
          [pltpu.SemaphoreType.DMA for _ in range(NBUF)],
          [pltpu.SemaphoreType.DMA for _ in range(NBUF)],
      ],
      compiler_params=pltpu.CompilerParams(use_tc_tiling_on_sc=False),
  )
  def scatter_kernel(table_hbm, src_hbm, dst_hbm, p_out, c_out,
                     sbuf, dbuf, rows, zrow, zcnt, ones,
                     acc_sh, cnt_sh, gsem, ssem):
    c = lax.axis_index("c")
    s = lax.axis_index("s")
    row0 = s * RPT

    zeros16 = jnp.zeros((16,), jnp.float32)
    ones16 = jnp.full((16,), 1.0, jnp.float32)

    def fill_zrow(i, carry):
      for j in range(DH // 16):
        zrow[i, pl.ds(j * 16, 16)] = zeros16
      return carry
    lax.fori_loop(0, 128, fill_zrow, 0)

    def fill_zcnt(i, carry):
      zcnt[pl.ds(i * 16, 16)] = zeros16
      return carry
    lax.fori_loop(0, RPT // 16, fill_zcnt, 0)

    for j in range(CH // 16):
      ones[pl.ds(j * 16, 16)] = ones16

    for k in range(RPT // 128):
      pltpu.sync_copy(zrow, acc_sh.at[pl.ds(row0 + k * 128, 128)])
    pltpu.sync_copy(zcnt, cnt_sh.at[pl.ds(row0, RPT)])

    plsc.subcore_barrier()

    pltpu.sync_copy(src_hbm.at[s], sbuf)
    pltpu.sync_copy(dst_hbm.at[s], dbuf)

    def g_start(k, j):
      pltpu.async_copy(table_hbm.at[c].at[sbuf.at[j]], rows[k], gsem[k])

    def g_wait(k):
      pltpu.make_async_copy(table_hbm.at[c].at[sbuf.at[0]], rows[k],
                            gsem[k]).wait()

    def s_start(k, j):
      pltpu.async_copy(rows[k], acc_sh.at[dbuf.at[j]], ssem[k], add=True)

    def s_wait(k):
      pltpu.make_async_copy(rows[k], acc_sh.at[dbuf.at[0]],
                            ssem[k]).wait()

    for k in range(NBUF):
      g_start(k, k)

    def body(t, carry):
      j0 = t * NBUF
      for k in range(NBUF):
        j = j0 + k
        g_wait(k)
        s_start(k, j)
        @pl.when((j % 2) == c)
        def _():
          pltpu.sync_copy(ones, cnt_sh.at[dbuf.at[j]], add=True)
      for k in range(NBUF):
        j2 = j0 + k + NBUF
        s_wait(k)
        @pl.when(j2 < NCHUNK)
        def _():
          g_start(k, j2)
      return carry
    lax.fori_loop(0, NCHUNK // NBUF, body, 0)

    plsc.subcore_barrier()

    pltpu.sync_copy(acc_sh.at[pl.ds(row0, RPT)],
                    p_out.at[c, pl.ds(row0, RPT)])
    pltpu.sync_copy(cnt_sh.at[pl.ds(row0, RPT)],
                    c_out.at[c, pl.ds(row0, RPT)])

  return scatter_kernel(table2, src_idx, dst_idx)


def _matmul_relu(X, WT, b2):
  blk = 1000

  def mm_body(x_ref, w_ref, b_ref, o_ref):
    h = jnp.dot(x_ref[...], w_ref[...], preferred_element_type=jnp.float32)
    h = jnp.maximum(h + b_ref[...], 0.0)
    o_ref[0] = h[:, :DH]
    o_ref[1] = h[:, DH:]

  return pl.pallas_call(
      mm_body,
      grid=(N_V // blk,),
      in_specs=[
          pl.BlockSpec((blk, D), lambda i: (i, 0)),
          pl.BlockSpec((D, D), lambda i: (0, 0)),
          pl.BlockSpec((1, D), lambda i: (0, 0)),
      ],
      out_specs=pl.BlockSpec((NC, blk, DH), lambda i: (0, i, 0)),
      out_shape=jax.ShapeDtypeStruct((NC, N_V, DH), jnp.float32),
  )(X, WT, b2)


def _combine_mean(P, C, relu, split_out):
  blk = 1024

  def comb_body(p_ref, c_ref, o_ref):
    d = jnp.maximum(c_ref[0] + c_ref[1], 1.0)
    r0 = p_ref[0] / d[:, None]
    r1 = p_ref[1] / d[:, None]
    if relu:
      r0 = jnp.maximum(r0, 0.0)
      r1 = jnp.maximum(r1, 0.0)
    if split_out:
      o_ref[0] = r0
      o_ref[1] = r1
    else:
      o_ref[:, :DH] = r0
      o_ref[:, DH:] = r1

  if split_out:
    out_spec = pl.BlockSpec((NC, blk, DH), lambda i: (0, i, 0))
    out_shape = jax.ShapeDtypeStruct((NC, RP, DH), jnp.float32)
  else:
    out_spec = pl.BlockSpec((blk, D), lambda i: (i, 0))
    out_shape = jax.ShapeDtypeStruct((RP, D), jnp.float32)

  return pl.pallas_call(
      comb_body,
      grid=(RP // blk,),
      in_specs=[
          pl.BlockSpec((NC, blk, DH), lambda i: (0, i, 0)),
          pl.BlockSpec((NC, blk), lambda i: (0, i)),
      ],
      out_specs=out_spec,
      out_shape=out_shape,
  )(P, C)


def kernel(X, W, b, edge_index, drop_rate):
  v = edge_index[0].astype(jnp.int32).reshape(NS, NCHUNK, CH)
  e = edge_index[1].astype(jnp.int32).reshape(NS, NCHUNK, CH)

  H2 = _matmul_relu(X, W.T, b.reshape(1, D))

  P1, C1 = _sc_scatter_pass(H2, v, e)
  Xe2 = _combine_mean(P1, C1, relu=False, split_out=True)

  P2, C2 = _sc_scatter_pass(Xe2, e, v)
  out = _combine_mean(P2, C2, relu=True, split_out=False)
  return out[:N_V]


# speedup vs baseline: 10.3838x; 1.1757x over previous
import functools

import jax
import jax.numpy as jnp
from jax import lax
from jax.experimental import pallas as pl
from jax.experimental.pallas import tpu as pltpu
from jax.experimental.pallas import tpu_sc as plsc

N_V = 10000
N_E = 10000
NNZ = 320000
D = 128
DH = D // 2

NC = 2
NS = 16

EPT = NNZ // NS
CH = 80
NCHUNK = EPT // CH
NBUF = 5

RP = 10240
RPT = RP // NS


def _sc_scatter_pass(table2, src_idx, dst_idx):
  mesh = plsc.VectorSubcoreMesh(
      core_axis_name="c", subcore_axis_name="s", num_cores=NC,
      num_subcores=NS)

  @functools.partial(
      pl.kernel,
      out_type=(
          jax.ShapeDtypeStruct((NC, RP, DH), jnp.float32),
          jax.ShapeDtypeStruct((NC, RP), jnp.float32),
      ),
      mesh=mesh,
      scratch_types=[
          pltpu.VMEM((NCHUNK, CH), jnp.int32),
          pltpu.VMEM((NCHUNK, CH), jnp.int32),
          [pltpu.VMEM((CH, DH), jnp.float32) for _ in range(NBUF)],
          pltpu.VMEM((128, DH), jnp.float32),
          pltpu.VMEM((RPT,), jnp.float32),
          pltpu.VMEM((CH,), jnp.float32),
          pltpu.VMEM_SHARED((RP, DH), jnp.float32),
          pltpu.VMEM_SHARED((RP,), jnp.float32),
          [pltpu.SemaphoreType.DMA for _ in range(NBUF)],
          [pltpu.SemaphoreType.DMA for _ in range(NBUF)],
          [pltpu.SemaphoreType.DMA for _ in range(NBUF)],
      ],
      compiler_params=pltpu.CompilerParams(use_tc_tiling_on_sc=False),
  )
  def scatter_kernel(table_hbm, src_hbm, dst_hbm, p_out, c_out,
                     sbuf, dbuf, rows, zrow, zcnt, ones,
                     acc_sh, cnt_sh, gsem, ssem, csem):
    c = lax.axis_index("c")
    s = lax.axis_index("s")
    row0 = s * RPT

    zeros16 = jnp.zeros((16,), jnp.float32)
    ones16 = jnp.full((16,), 1.0, jnp.float32)

    def fill_zrow(i, carry):
      for j in range(DH // 16):
        zrow[i, pl.ds(j * 16, 16)] = zeros16
      return carry
    lax.fori_loop(0, 128, fill_zrow, 0)

    def fill_zcnt(i, carry):
      zcnt[pl.ds(i * 16, 16)] = zeros16
      return carry
    lax.fori_loop(0, RPT // 16, fill_zcnt, 0)

    for j in range(CH // 16):
      ones[pl.ds(j * 16, 16)] = ones16

    for k in range(RPT // 128):
      pltpu.sync_copy(zrow, acc_sh.at[pl.ds(row0 + k * 128, 128)])
    pltpu.sync_copy(zcnt, cnt_sh.at[pl.ds(row0, RPT)])

    plsc.subcore_barrier()

    pltpu.sync_copy(src_hbm.at[s], sbuf)
    pltpu.sync_copy(dst_hbm.at[s], dbuf)

    def g_start(k, j):
      pltpu.async_copy(table_hbm.at[c].at[sbuf.at[j]], rows[k], gsem[k])

    def g_wait(k):
      pltpu.make_async_copy(table_hbm.at[c].at[sbuf.at[0]], rows[k],
                            gsem[k]).wait()

    def s_start(k, j):
      pltpu.async_copy(rows[k], acc_sh.at[dbuf.at[j]], ssem[k], add=True)

    def s_wait(k):
      pltpu.make_async_copy(rows[k], acc_sh.at[dbuf.at[0]],
                            ssem[k]).wait()

    for k in range(NBUF):
      g_start(k, k)

    def c_wait(k):
      pltpu.make_async_copy(ones, cnt_sh.at[dbuf.at[0]], csem[k]).wait()

    def body(t, carry):
      j0 = t * NBUF
      for k in range(NBUF):
        j = j0 + k
        g_wait(k)
        s_start(k, j)
        @pl.when((j % 2) == c)
        def _():
          pltpu.async_copy(ones, cnt_sh.at[dbuf.at[j]], csem[k], add=True)
      for k in range(NBUF):
        j = j0 + k
        j2 = j + NBUF
        s_wait(k)
        @pl.when((j % 2) == c)
        def _():
          c_wait(k)
        @pl.when(j2 < NCHUNK)
        def _():
          g_start(k, j2)
      return carry
    lax.fori_loop(0, NCHUNK // NBUF, body, 0)

    plsc.subcore_barrier()

    pltpu.sync_copy(acc_sh.at[pl.ds(row0, RPT)],
                    p_out.at[c, pl.ds(row0, RPT)])
    pltpu.sync_copy(cnt_sh.at[pl.ds(row0, RPT)],
                    c_out.at[c, pl.ds(row0, RPT)])

  return scatter_kernel(table2, src_idx, dst_idx)


def _matmul_relu(X, WT, b2):
  blk = 1000

  def mm_body(x_ref, w_ref, b_ref, o_ref):
    h = jnp.dot(x_ref[...], w_ref[...], preferred_element_type=jnp.float32)
    h = jnp.maximum(h + b_ref[...], 0.0)
    o_ref[0] = h[:, :DH]
    o_ref[1] = h[:, DH:]

  return pl.pallas_call(
      mm_body,
      grid=(N_V // blk,),
      in_specs=[
          pl.BlockSpec((blk, D), lambda i: (i, 0)),
          pl.BlockSpec((D, D), lambda i: (0, 0)),
          pl.BlockSpec((1, D), lambda i: (0, 0)),
      ],
      out_specs=pl.BlockSpec((NC, blk, DH), lambda i: (0, i, 0)),
      out_shape=jax.ShapeDtypeStruct((NC, N_V, DH), jnp.float32),
  )(X, WT, b2)


def _combine_mean(P, C, relu, split_out):
  blk = 1024

  def comb_body(p_ref, c_ref, o_ref):
    d = jnp.maximum(c_ref[0] + c_ref[1], 1.0)
    r0 = p_ref[0] / d[:, None]
    r1 = p_ref[1] / d[:, None]
    if relu:
      r0 = jnp.maximum(r0, 0.0)
      r1 = jnp.maximum(r1, 0.0)
    if split_out:
      o_ref[0] = r0
      o_ref[1] = r1
    else:
      o_ref[:, :DH] = r0
      o_ref[:, DH:] = r1

  if split_out:
    out_spec = pl.BlockSpec((NC, blk, DH), lambda i: (0, i, 0))
    out_shape = jax.ShapeDtypeStruct((NC, RP, DH), jnp.float32)
  else:
    out_spec = pl.BlockSpec((blk, D), lambda i: (i, 0))
    out_shape = jax.ShapeDtypeStruct((RP, D), jnp.float32)

  return pl.pallas_call(
      comb_body,
      grid=(RP // blk,),
      in_specs=[
          pl.BlockSpec((NC, blk, DH), lambda i: (0, i, 0)),
          pl.BlockSpec((NC, blk), lambda i: (0, i)),
      ],
      out_specs=out_spec,
      out_shape=out_shape,
  )(P, C)


def kernel(X, W, b, edge_index, drop_rate):
  v = edge_index[0].astype(jnp.int32).reshape(NS, NCHUNK, CH)
  e = edge_index[1].astype(jnp.int32).reshape(NS, NCHUNK, CH)

  H2 = _matmul_relu(X, W.T, b.reshape(1, D))

  P1, C1 = _sc_scatter_pass(H2, v, e)
  Xe2 = _combine_mean(P1, C1, relu=False, split_out=True)

  P2, C2 = _sc_scatter_pass(Xe2, e, v)
  out = _combine_mean(P2, C2, relu=True, split_out=False)
  return out[:N_V]


# speedup vs baseline: 10.4640x; 1.0077x over previous
import functools

import jax
import jax.numpy as jnp
from jax import lax
from jax.experimental import pallas as pl
from jax.experimental.pallas import tpu as pltpu
from jax.experimental.pallas import tpu_sc as plsc

N_V = 10000
N_E = 10000
NNZ = 320000
D = 128
DH = D // 2

NC = 2
NS = 16

CH = 128
NCHUNK = 160
NBUF = 5
NNZ_PAD = NS * NCHUNK * CH - NNZ

RP = 10240
RPT = RP // NS


def _sc_scatter_pass(table2, src_idx, dst_idx):
  mesh = plsc.VectorSubcoreMesh(
      core_axis_name="c", subcore_axis_name="s", num_cores=NC,
      num_subcores=NS)

  @functools.partial(
      pl.kernel,
      out_type=(
          jax.ShapeDtypeStruct((NC, RP, DH), jnp.float32),
          jax.ShapeDtypeStruct((NC, RP), jnp.float32),
      ),
      mesh=mesh,
      scratch_types=[
          pltpu.VMEM((NCHUNK, CH), jnp.int32),
          pltpu.VMEM((NCHUNK, CH), jnp.int32),
          [pltpu.VMEM((CH, DH), jnp.float32) for _ in range(NBUF)],
          pltpu.VMEM((32, DH), jnp.float32),
          pltpu.VMEM((RPT,), jnp.float32),
          pltpu.VMEM((CH,), jnp.float32),
          pltpu.VMEM_SHARED((RP, DH), jnp.float32),
          pltpu.VMEM_SHARED((RP,), jnp.float32),
          [pltpu.SemaphoreType.DMA for _ in range(NBUF)],
          [pltpu.SemaphoreType.DMA for _ in range(NBUF)],
          [pltpu.SemaphoreType.DMA for _ in range(NBUF)],
      ],
      compiler_params=pltpu.CompilerParams(use_tc_tiling_on_sc=False),
  )
  def scatter_kernel(table_hbm, src_hbm, dst_hbm, p_out, c_out,
                     sbuf, dbuf, rows, zrow, zcnt, ones,
                     acc_sh, cnt_sh, gsem, ssem, csem):
    c = lax.axis_index("c")
    s = lax.axis_index("s")
    row0 = s * RPT

    zeros16 = jnp.zeros((16,), jnp.float32)
    ones16 = jnp.full((16,), 1.0, jnp.float32)

    def fill_zrow(i, carry):
      for j in range(DH // 16):
        zrow[i, pl.ds(j * 16, 16)] = zeros16
      return carry
    lax.fori_loop(0, 32, fill_zrow, 0)

    def fill_zcnt(i, carry):
      zcnt[pl.ds(i * 16, 16)] = zeros16
      return carry
    lax.fori_loop(0, RPT // 16, fill_zcnt, 0)

    for j in range(CH // 16):
      ones[pl.ds(j * 16, 16)] = ones16

    for k in range(RPT // 32):
      pltpu.sync_copy(zrow, acc_sh.at[pl.ds(row0 + k * 32, 32)])
    pltpu.sync_copy(zcnt, cnt_sh.at[pl.ds(row0, RPT)])

    plsc.subcore_barrier()

    pltpu.sync_copy(src_hbm.at[s], sbuf)
    pltpu.sync_copy(dst_hbm.at[s], dbuf)

    def g_start(k, j):
      pltpu.async_copy(table_hbm.at[c].at[sbuf.at[j]], rows[k], gsem[k])

    def g_wait(k):
      pltpu.make_async_copy(table_hbm.at[c].at[sbuf.at[0]], rows[k],
                            gsem[k]).wait()

    def s_start(k, j):
      pltpu.async_copy(rows[k], acc_sh.at[dbuf.at[j]], ssem[k], add=True)

    def s_wait(k):
      pltpu.make_async_copy(rows[k], acc_sh.at[dbuf.at[0]],
                            ssem[k]).wait()

    for k in range(NBUF):
      g_start(k, k)

    def c_wait(k):
      pltpu.make_async_copy(ones, cnt_sh.at[dbuf.at[0]], csem[k]).wait()

    def body(t, carry):
      j0 = t * NBUF
      for k in range(NBUF):
        j = j0 + k
        g_wait(k)
        s_start(k, j)
        @pl.when((j % 2) == c)
        def _():
          pltpu.async_copy(ones, cnt_sh.at[dbuf.at[j]], csem[k], add=True)
      for k in range(NBUF):
        j = j0 + k
        j2 = j + NBUF
        s_wait(k)
        @pl.when((j % 2) == c)
        def _():
          c_wait(k)
        @pl.when(j2 < NCHUNK)
        def _():
          g_start(k, j2)
      return carry
    lax.fori_loop(0, NCHUNK // NBUF, body, 0)

    plsc.subcore_barrier()

    pltpu.sync_copy(acc_sh.at[pl.ds(row0, RPT)],
                    p_out.at[c, pl.ds(row0, RPT)])
    pltpu.sync_copy(cnt_sh.at[pl.ds(row0, RPT)],
                    c_out.at[c, pl.ds(row0, RPT)])

  return scatter_kernel(table2, src_idx, dst_idx)


def _matmul_relu(X, WT, b2):
  blk = 1000

  def mm_body(x_ref, w_ref, b_ref, o_ref):
    h = jnp.dot(x_ref[...], w_ref[...], preferred_element_type=jnp.float32)
    h = jnp.maximum(h + b_ref[...], 0.0)
    o_ref[0] = h[:, :DH]
    o_ref[1] = h[:, DH:]

  return pl.pallas_call(
      mm_body,
      grid=(N_V // blk,),
      in_specs=[
          pl.BlockSpec((blk, D), lambda i: (i, 0)),
          pl.BlockSpec((D, D), lambda i: (0, 0)),
          pl.BlockSpec((1, D), lambda i: (0, 0)),
      ],
      out_specs=pl.BlockSpec((NC, blk, DH), lambda i: (0, i, 0)),
      out_shape=jax.ShapeDtypeStruct((NC, RP, DH), jnp.float32),
  )(X, WT, b2)


def _combine_mean(P, C, relu, split_out):
  blk = 1024

  def comb_body(p_ref, c_ref, o_ref):
    d = jnp.maximum(c_ref[0] + c_ref[1], 1.0)
    r0 = p_ref[0] / d[:, None]
    r1 = p_ref[1] / d[:, None]
    if relu:
      r0 = jnp.maximum(r0, 0.0)
      r1 = jnp.maximum(r1, 0.0)
    if split_out:
      o_ref[0] = r0
      o_ref[1] = r1
    else:
      o_ref[:, :DH] = r0
      o_ref[:, DH:] = r1

  if split_out:
    out_spec = pl.BlockSpec((NC, blk, DH), lambda i: (0, i, 0))
    out_shape = jax.ShapeDtypeStruct((NC, RP, DH), jnp.float32)
  else:
    out_spec = pl.BlockSpec((blk, D), lambda i: (i, 0))
    out_shape = jax.ShapeDtypeStruct((RP, D), jnp.float32)

  return pl.pallas_call(
      comb_body,
      grid=(RP // blk,),
      in_specs=[
          pl.BlockSpec((NC, blk, DH), lambda i: (0, i, 0)),
          pl.BlockSpec((NC, blk), lambda i: (0, i)),
      ],
      out_specs=out_spec,
      out_shape=out_shape,
  )(P, C)


def kernel(X, W, b, edge_index, drop_rate):
  pad = N_V + (jnp.arange(NNZ_PAD, dtype=jnp.int32) % (RP - N_V))
  v = jnp.concatenate([edge_index[0].astype(jnp.int32), pad])
  e = jnp.concatenate([edge_index[1].astype(jnp.int32), pad])
  v = v.reshape(NS, NCHUNK, CH)
  e = e.reshape(NS, NCHUNK, CH)

  H2 = _matmul_relu(X, W.T, b.reshape(1, D))

  P1, C1 = _sc_scatter_pass(H2, v, e)
  Xe2 = _combine_mean(P1, C1, relu=False, split_out=True)

  P2, C2 = _sc_scatter_pass(Xe2, e, v)
  out = _combine_mean(P2, C2, relu=True, split_out=False)
  return out[:N_V]


# speedup vs baseline: 11.3125x; 1.0811x over previous
import functools

import jax
import jax.numpy as jnp
from jax import lax
from jax.experimental import pallas as pl
from jax.experimental.pallas import tpu as pltpu
from jax.experimental.pallas import tpu_sc as plsc

N_V = 10000
N_E = 10000
NNZ = 320000
D = 128
DH = D // 2

NC = 2
NS = 16

CH = 128
NCHUNK = 160
NBUF = 5
NNZ_PAD = NS * NCHUNK * CH - NNZ

RP = 10240
RPT = RP // NS


def _sc_scatter_pass(table2, src_idx, dst_idx, relu):
  mesh = plsc.VectorSubcoreMesh(
      core_axis_name="c", subcore_axis_name="s", num_cores=NC,
      num_subcores=NS)

  @functools.partial(
      pl.kernel,
      out_type=jax.ShapeDtypeStruct((NC, RP, DH), jnp.float32),
      mesh=mesh,
      scratch_types=[
          pltpu.VMEM((NCHUNK, CH), jnp.int32),
          pltpu.VMEM((NCHUNK, CH), jnp.int32),
          [pltpu.VMEM((CH, DH), jnp.float32) for _ in range(NBUF)],
          pltpu.VMEM((32, DH), jnp.float32),
          pltpu.VMEM((RPT,), jnp.float32),
          pltpu.VMEM((CH,), jnp.float32),
          pltpu.VMEM_SHARED((RP, DH), jnp.float32),
          pltpu.VMEM_SHARED((RP,), jnp.float32),
          [pltpu.SemaphoreType.DMA for _ in range(NBUF)],
          [pltpu.SemaphoreType.DMA for _ in range(NBUF)],
          [pltpu.SemaphoreType.DMA for _ in range(NBUF)],
      ],
      compiler_params=pltpu.CompilerParams(use_tc_tiling_on_sc=False),
  )
  def scatter_kernel(table_hbm, src_hbm, dst_hbm, p_out,
                     sbuf, dbuf, rows, zrow, cntv, ones,
                     acc_sh, cnt_sh, gsem, ssem, csem):
    c = lax.axis_index("c")
    s = lax.axis_index("s")
    row0 = s * RPT

    zeros16 = jnp.zeros((16,), jnp.float32)
    ones16 = jnp.full((16,), 1.0, jnp.float32)

    def fill_zrow(i, carry):
      for j in range(DH // 16):
        zrow[i, pl.ds(j * 16, 16)] = zeros16
      return carry
    lax.fori_loop(0, 32, fill_zrow, 0)

    def fill_zcnt(i, carry):
      cntv[pl.ds(i * 16, 16)] = zeros16
      return carry
    lax.fori_loop(0, RPT // 16, fill_zcnt, 0)

    for j in range(CH // 16):
      ones[pl.ds(j * 16, 16)] = ones16

    for k in range(RPT // 32):
      pltpu.sync_copy(zrow, acc_sh.at[pl.ds(row0 + k * 32, 32)])
    pltpu.sync_copy(cntv, cnt_sh.at[pl.ds(row0, RPT)])

    plsc.subcore_barrier()

    pltpu.sync_copy(src_hbm.at[s], sbuf)
    pltpu.sync_copy(dst_hbm.at[s], dbuf)

    def g_start(k, j):
      pltpu.async_copy(table_hbm.at[c].at[sbuf.at[j]], rows[k], gsem[k])

    def g_wait(k):
      pltpu.make_async_copy(table_hbm.at[c].at[sbuf.at[0]], rows[k],
                            gsem[k]).wait()

    def s_start(k, j):
      pltpu.async_copy(rows[k], acc_sh.at[dbuf.at[j]], ssem[k], add=True)

    def s_wait(k):
      pltpu.make_async_copy(rows[k], acc_sh.at[dbuf.at[0]],
                            ssem[k]).wait()

    def c_wait(k):
      pltpu.make_async_copy(ones, cnt_sh.at[dbuf.at[0]], csem[k]).wait()

    for k in range(NBUF):
      g_start(k, k)

    def body(t, carry):
      j0 = t * NBUF
      for k in range(NBUF):
        j = j0 + k
        g_wait(k)
        s_start(k, j)
        pltpu.async_copy(ones, cnt_sh.at[dbuf.at[j]], csem[k], add=True)
      for k in range(NBUF):
        j2 = j0 + k + NBUF
        s_wait(k)
        c_wait(k)
        @pl.when(j2 < NCHUNK)
        def _():
          g_start(k, j2)
      return carry
    lax.fori_loop(0, NCHUNK // NBUF, body, 0)

    plsc.subcore_barrier()

    pltpu.sync_copy(cnt_sh.at[pl.ds(row0, RPT)], cntv)
    for k in range(RPT // CH):
      pltpu.sync_copy(acc_sh.at[pl.ds(row0 + k * CH, CH)], rows[0])

      def div_group(g, carry):
        cv = cntv[pl.ds(k * CH + g * 16, 16)]
        rcpv = 1.0 / jnp.maximum(cv, 1.0)
        for rl in range(16):
          rcp = rcpv[rl]
          r = g * 16 + rl
          for q in range(DH // 16):
            x = rows[0][r, pl.ds(q * 16, 16)] * rcp
            if relu:
              x = jnp.maximum(x, 0.0)
            rows[0][r, pl.ds(q * 16, 16)] = x
        return carry
      lax.fori_loop(0, CH // 16, div_group, 0)

      pltpu.sync_copy(rows[0], p_out.at[c, pl.ds(row0 + k * CH, CH)])

  return scatter_kernel(table2, src_idx, dst_idx)


def _matmul_relu(X, WT, b2):
  blk = 1000

  def mm_body(x_ref, w_ref, b_ref, o_ref):
    h = jnp.dot(x_ref[...], w_ref[...], preferred_element_type=jnp.float32)
    h = jnp.maximum(h + b_ref[...], 0.0)
    o_ref[0] = h[:, :DH]
    o_ref[1] = h[:, DH:]

  return pl.pallas_call(
      mm_body,
      grid=(N_V // blk,),
      in_specs=[
          pl.BlockSpec((blk, D), lambda i: (i, 0)),
          pl.BlockSpec((D, D), lambda i: (0, 0)),
          pl.BlockSpec((1, D), lambda i: (0, 0)),
      ],
      out_specs=pl.BlockSpec((NC, blk, DH), lambda i: (0, i, 0)),
      out_shape=jax.ShapeDtypeStruct((NC, RP, DH), jnp.float32),
  )(X, WT, b2)


def kernel(X, W, b, edge_index, drop_rate):
  pad = N_V + (jnp.arange(NNZ_PAD, dtype=jnp.int32) % (RP - N_V))
  v = jnp.concatenate([edge_index[0].astype(jnp.int32), pad])
  e = jnp.concatenate([edge_index[1].astype(jnp.int32), pad])
  v = v.reshape(NS, NCHUNK, CH)
  e = e.reshape(NS, NCHUNK, CH)

  H2 = _matmul_relu(X, W.T, b.reshape(1, D))

  Xe2 = _sc_scatter_pass(H2, v, e, relu=False)
  P2 = _sc_scatter_pass(Xe2, e, v, relu=True)

  return jnp.concatenate([P2[0], P2[1]], axis=-1)[:N_V]


# speedup vs baseline: 11.6423x; 1.0291x over previous
import functools

import jax
import jax.numpy as jnp
from jax import lax
from jax.experimental import pallas as pl
from jax.experimental.pallas import tpu as pltpu
from jax.experimental.pallas import tpu_sc as plsc

N_V = 10000
N_E = 10000
NNZ = 320000
D = 128
DH = D // 2

NC = 2
NS = 16

CH = 128
NCHUNK = 160
NBUF = 5
NNZ_PAD = NS * NCHUNK * CH - NNZ

RP = 10240
RPT = RP // NS


def _sc_v2v_pass(table2, v_idx, e_idx):
  mesh = plsc.VectorSubcoreMesh(
      core_axis_name="c", subcore_axis_name="s", num_cores=NC,
      num_subcores=NS)

  @functools.partial(
      pl.kernel,
      out_type=(
          jax.ShapeDtypeStruct((NC, RP, DH), jnp.float32),
          jax.ShapeDtypeStruct((NC, RP, DH), jnp.float32),
      ),
      mesh=mesh,
      scratch_types=[
          pltpu.VMEM((NCHUNK, CH), jnp.int32),
          pltpu.VMEM((NCHUNK, CH), jnp.int32),
          [pltpu.VMEM((CH, DH), jnp.float32) for _ in range(NBUF)],
          pltpu.VMEM((32, DH), jnp.float32),
          pltpu.VMEM((RPT,), jnp.float32),
          pltpu.VMEM((CH,), jnp.float32),
          pltpu.VMEM_SHARED((RP, DH), jnp.float32),
          pltpu.VMEM_SHARED((RP,), jnp.float32),
          [pltpu.SemaphoreType.DMA for _ in range(NBUF)],
          [pltpu.SemaphoreType.DMA for _ in range(NBUF)],
          [pltpu.SemaphoreType.DMA for _ in range(NBUF)],
      ],
      compiler_params=pltpu.CompilerParams(use_tc_tiling_on_sc=False),
  )
  def v2v_kernel(table_hbm, v_hbm, e_hbm, xe_out, p_out,
                 vbuf, ebuf, rows, zrow, cntv, ones,
                 acc_sh, cnt_sh, gsem, ssem, csem):
    c = lax.axis_index("c")
    s = lax.axis_index("s")
    row0 = s * RPT

    zeros16 = jnp.zeros((16,), jnp.float32)
    ones16 = jnp.full((16,), 1.0, jnp.float32)

    def fill_zrow(i, carry):
      for j in range(DH // 16):
        zrow[i, pl.ds(j * 16, 16)] = zeros16
      return carry
    lax.fori_loop(0, 32, fill_zrow, 0)

    def fill_zcnt(i, carry):
      cntv[pl.ds(i * 16, 16)] = zeros16
      return carry

    for j in range(CH // 16):
      ones[pl.ds(j * 16, 16)] = ones16

    def zero_acc():
      for k in range(RPT // 32):
        pltpu.sync_copy(zrow, acc_sh.at[pl.ds(row0 + k * 32, 32)])
      lax.fori_loop(0, RPT // 16, fill_zcnt, 0)
      pltpu.sync_copy(cntv, cnt_sh.at[pl.ds(row0, RPT)])

    zero_acc()
    plsc.subcore_barrier()

    pltpu.sync_copy(v_hbm.at[s], vbuf)
    pltpu.sync_copy(e_hbm.at[s], ebuf)

    def run_stage(src_table, gidx, sidx):
      def g_start(k, j):
        pltpu.async_copy(src_table.at[c].at[gidx.at[j]], rows[k], gsem[k])

      def g_wait(k):
        pltpu.make_async_copy(src_table.at[c].at[gidx.at[0]], rows[k],
                              gsem[k]).wait()

      def s_start(k, j):
        pltpu.async_copy(rows[k], acc_sh.at[sidx.at[j]], ssem[k], add=True)

      def s_wait(k):
        pltpu.make_async_copy(rows[k], acc_sh.at[sidx.at[0]],
                              ssem[k]).wait()

      def c_wait(k):
        pltpu.make_async_copy(ones, cnt_sh.at[sidx.at[0]], csem[k]).wait()

      for k in range(NBUF):
        g_start(k, k)

      def body(t, carry):
        j0 = t * NBUF
        for k in range(NBUF):
          j = j0 + k
          g_wait(k)
          s_start(k, j)
          pltpu.async_copy(ones, cnt_sh.at[sidx.at[j]], csem[k], add=True)
        for k in range(NBUF):
          j2 = j0 + k + NBUF
          s_wait(k)
          c_wait(k)
          @pl.when(j2 < NCHUNK)
          def _():
            g_start(k, j2)
        return carry
      lax.fori_loop(0, NCHUNK // NBUF, body, 0)

    def divide_writeout(dst, relu):
      pltpu.sync_copy(cnt_sh.at[pl.ds(row0, RPT)], cntv)
      for k in range(RPT // CH):
        pltpu.sync_copy(acc_sh.at[pl.ds(row0 + k * CH, CH)], rows[0])

        def div_group(g, carry):
          cv = cntv[pl.ds(k * CH + g * 16, 16)]
          rcpv = 1.0 / jnp.maximum(cv, 1.0)
          for rl in range(16):
            rcp = rcpv[rl]
            r = g * 16 + rl
            for q in range(DH // 16):
              x = rows[0][r, pl.ds(q * 16, 16)] * rcp
              if relu:
                x = jnp.maximum(x, 0.0)
              rows[0][r, pl.ds(q * 16, 16)] = x
          return carry
        lax.fori_loop(0, CH // 16, div_group, 0)

        pltpu.sync_copy(rows[0], dst.at[c, pl.ds(row0 + k * CH, CH)])

    run_stage(table_hbm, vbuf, ebuf)
    plsc.subcore_barrier()
    divide_writeout(xe_out, relu=False)
    zero_acc()
    plsc.subcore_barrier()

    run_stage(xe_out, ebuf, vbuf)
    plsc.subcore_barrier()
    divide_writeout(p_out, relu=True)

  return v2v_kernel(table2, v_idx, e_idx)


def _matmul_relu(X, WT, b2):
  blk = 1000

  def mm_body(x_ref, w_ref, b_ref, o_ref):
    h = jnp.dot(x_ref[...], w_ref[...], preferred_element_type=jnp.float32)
    h = jnp.maximum(h + b_ref[...], 0.0)
    o_ref[0] = h[:, :DH]
    o_ref[1] = h[:, DH:]

  return pl.pallas_call(
      mm_body,
      grid=(N_V // blk,),
      in_specs=[
          pl.BlockSpec((blk, D), lambda i: (i, 0)),
          pl.BlockSpec((D, D), lambda i: (0, 0)),
          pl.BlockSpec((1, D), lambda i: (0, 0)),
      ],
      out_specs=pl.BlockSpec((NC, blk, DH), lambda i: (0, i, 0)),
      out_shape=jax.ShapeDtypeStruct((NC, RP, DH), jnp.float32),
  )(X, WT, b2)


def kernel(X, W, b, edge_index, drop_rate):
  pad = N_V + (jnp.arange(NNZ_PAD, dtype=jnp.int32) % (RP - N_V))
  v = jnp.concatenate([edge_index[0].astype(jnp.int32), pad])
  e = jnp.concatenate([edge_index[1].astype(jnp.int32), pad])
  v = v.reshape(NS, NCHUNK, CH)
  e = e.reshape(NS, NCHUNK, CH)

  H2 = _matmul_relu(X, W.T, b.reshape(1, D))

  _, P2 = _sc_v2v_pass(H2, v, e)

  return jnp.concatenate([P2[0], P2[1]], axis=-1)[:N_V]
